# baseline (speedup 1.0000x reference)
import numpy as np
import jax
import jax.numpy as jnp
from jax.experimental import pallas as pl
from jax.experimental.pallas import tpu as pltpu

N = 10000
D = 128
H = 128
C = 2
BM = 400
NB = N // BM
BW = 1280
NBAND = 8
LASTW = N - (NBAND - 1) * BW
N_PAD = NBAND * BW


def _make_schedule():
    r1 = np.repeat(np.arange(NB), NBAND)
    c1 = np.tile(np.arange(NBAND), NB)
    r2, c2 = [], []
    for j in range(NB):
        for b in range(NBAND):
            if min((b + 1) * BW, N) > j * BM:
                r2.append(j)
                c2.append(b)
    rows = np.concatenate([r1, np.array(r2)]).astype(np.int32)
    cols = np.concatenate([c1, np.array(c2)]).astype(np.int32)
    s1 = len(r1)
    orow = np.concatenate([np.zeros(s1, np.int64), np.array(r2)]).astype(np.int32)
    return rows, cols, orow, s1, s1 + len(r2)


_ROWS, _COLS, _OROW, S1, S = _make_schedule()


def _uni_kernel(rows_ref, cols_ref, orow_ref, adj_ref, feat_ref, g_ref, be_ref,
                w1_ref, b1_ref, a1_ref, w2_ref, b2_ref, a2_ref, w3_ref, b3_ref,
                o_ref, xn_s, x1_s, x2_s):
    s = pl.program_id(0)
    j = rows_ref[s]
    b = cols_ref[s]

    @pl.when(s == 0)
    def _init():
        x = feat_ref[...]
        mu = jnp.mean(x, axis=0, keepdims=True)
        var = jnp.mean((x - mu) * (x - mu), axis=0, keepdims=True)
        xn = (x - mu) * jax.lax.rsqrt(var + 1e-5) * g_ref[...] + be_ref[...]
        xn_s[0:N, :] = xn
        xn_s[N:N_PAD, :] = jnp.zeros((N_PAD - N, D), jnp.float32)
        x1_s[...] = jnp.zeros((N_PAD, D), jnp.float32)
        x2_s[...] = jnp.zeros((N, D), jnp.float32)

    a = adj_ref[...]
    col = jax.lax.broadcasted_iota(jnp.int32, (BM, BW), 1)
    a = jnp.where((b < NBAND - 1) | (col < LASTW), a, 0.0)

    @pl.when(s < S1)
    def _phase1():
        xk = xn_s[pl.ds(b * BW, BW), :]
        x1_s[pl.ds(j * BM, BM), :] += jnp.dot(
            a, xk, preferred_element_type=jnp.float32)

        @pl.when((b + 1) * BW <= j * BM)
        def _fused_lower():
            x1k = x1_s[pl.ds(b * BW, BW), :]
            x2_s[pl.ds(j * BM, BM), :] += jnp.dot(
                a, x1k, preferred_element_type=jnp.float32)

    @pl.when(s >= S1)
    def _phase2():
        x1k = x1_s[pl.ds(b * BW, BW), :]
        x2_s[pl.ds(j * BM, BM), :] += jnp.dot(
            a, x1k, preferred_element_type=jnp.float32)

        @pl.when(b == NBAND - 1)
        def _head():
            x2 = x2_s[pl.ds(j * BM, BM), :]
            h = jnp.dot(x2, w1_ref[...],
                        preferred_element_type=jnp.float32) + b1_ref[...]
            h = jnp.where(h >= 0, h, a1_ref[0, 0] * h)
            h = jnp.dot(h, w2_ref[...],
                        preferred_element_type=jnp.float32) + b2_ref[...]
            h = jnp.where(h >= 0, h, a2_ref[0, 0] * h)
            h = jnp.dot(h, w3_ref[...],
                        preferred_element_type=jnp.float32) + b3_ref[...]
            m = jnp.max(h, axis=1, keepdims=True)
            sh = h - m
            lse = jnp.log(jnp.sum(jnp.exp(sh), axis=1, keepdims=True))
            o_ref[...] = sh - lse


def kernel(feature, adj, gamma, beta, W1, b1, a1, W2, b2, a2, W3, b3):
    const = lambda s, r, c, o: (0, 0)
    grid_spec = pltpu.PrefetchScalarGridSpec(
        num_scalar_prefetch=3,
        grid=(S,),
        in_specs=[
            pl.BlockSpec((BM, BW), lambda s, r, c, o: (r[s], c[s])),
            pl.BlockSpec((N, D), const),
            pl.BlockSpec((1, D), const),
            pl.BlockSpec((1, D), const),
            pl.BlockSpec((D, H), const),
            pl.BlockSpec((1, H), const),
            pl.BlockSpec((1, 1), const),
            pl.BlockSpec((H, H), const),
            pl.BlockSpec((1, H), const),
            pl.BlockSpec((1, 1), const),
            pl.BlockSpec((H, C), const),
            pl.BlockSpec((1, C), const),
        ],
        out_specs=pl.BlockSpec((BM, C), lambda s, r, c, o: (o[s], 0)),
        scratch_shapes=[
            pltpu.VMEM((N_PAD, D), jnp.float32),
            pltpu.VMEM((N_PAD, D), jnp.float32),
            pltpu.VMEM((N, D), jnp.float32),
        ],
    )
    return pl.pallas_call(
        _uni_kernel,
        grid_spec=grid_spec,
        out_shape=jax.ShapeDtypeStruct((N, C), jnp.float32),
    )(jnp.asarray(_ROWS), jnp.asarray(_COLS), jnp.asarray(_OROW),
      adj, feature, gamma.reshape(1, D), beta.reshape(1, D),
      W1, b1.reshape(1, H), a1.reshape(1, 1),
      W2, b2.reshape(1, H), a2.reshape(1, 1),
      W3, b3.reshape(1, C))
